# baseline (speedup 1.0000x reference)
import functools

import jax
import jax.numpy as jnp
from jax import lax
from jax.experimental import pallas as pl
from jax.experimental.pallas import tpu as pltpu
from jax.experimental.pallas import tpu_sc as plsc

N = 10000
E = 160000
D = 256
H = 128
TECS = 16
EPT = E // TECS
CH = 125
CHP = 128
NCH = EPT // CH
ROWS2D = E // CH
SNPT = 632
SNPL = N - 15 * SNPT
NPAD = 10240
DPT = NPAD // TECS

_mesh = plsc.VectorSubcoreMesh(core_axis_name="c", subcore_axis_name="s")

_f32 = jnp.float32
_i32 = jnp.int32


def _deg_body(dst1, ew1, dst2, ew2, out, dst_v, ew_v, zbuf, acc):
    c = lax.axis_index("c")
    s = lax.axis_index("s")

    @pl.when(c == 0)
    def _():
        pltpu.sync_copy(dst1.at[pl.ds(s * NCH, NCH)], dst_v)
        pltpu.sync_copy(ew1.at[pl.ds(s * NCH, NCH)], ew_v)

    @pl.when(c == 1)
    def _():
        pltpu.sync_copy(dst2.at[pl.ds(s * NCH, NCH)], dst_v)
        pltpu.sync_copy(ew2.at[pl.ds(s * NCH, NCH)], ew_v)

    def _zero(i, _):
        zbuf[pl.ds(i * 16, 16)] = jnp.zeros((16,), _f32)
        return 0

    lax.fori_loop(0, DPT // 16, _zero, 0)
    pltpu.sync_copy(zbuf, acc.at[pl.ds(s * DPT, DPT)])
    plsc.subcore_barrier()

    def _chunk(ci, _):
        pltpu.sync_copy(ew_v.at[ci], acc.at[dst_v.at[ci]], add=True)
        return 0

    lax.fori_loop(0, NCH, _chunk, 0)
    plsc.subcore_barrier()

    @pl.when(c == 0)
    def _():
        pltpu.sync_copy(acc.at[pl.ds(s * DPT, DPT)], out.at[0, pl.ds(s * DPT, DPT)])

    @pl.when(c == 1)
    def _():
        pltpu.sync_copy(acc.at[pl.ds(s * DPT, DPT)], out.at[1, pl.ds(s * DPT, DPT)])


_deg_call = functools.partial(
    pl.kernel,
    out_type=jax.ShapeDtypeStruct((2, NPAD), _f32),
    mesh=_mesh,
    scratch_types=[
        pltpu.VMEM((NCH, CH), _i32),
        pltpu.VMEM((NCH, CH), _f32),
        pltpu.VMEM((DPT,), _f32),
        pltpu.VMEM_SHARED((NPAD,), _f32),
    ],
)(_deg_body)


def _seg_body(packed3d, ew3d, g0, g1, out0, out1,
              ring_p, ring_e, src_r, dst_r, buf_a, buf_b, buf_c, acc,
              sg0, sg1, sg2, ss0, ss1, ss2, si0, si1, si2):
    c = lax.axis_index("c")
    s = lax.axis_index("s")
    bufs = (buf_a, buf_b, buf_c)
    sg = (sg0, sg1, sg2)
    ss = (ss0, ss1, ss2)
    si = (si0, si1, si2)

    def _zrow(r, _):
        for j in range(H // 16):
            buf_a[r, pl.ds(j * 16, 16)] = jnp.zeros((16,), _f32)
        return 0

    lax.fori_loop(0, CHP, _zrow, 0)

    @pl.when(s < 15)
    def _():
        off = 0
        for sz in (128, 128, 128, 128, 120):
            pltpu.sync_copy(buf_a.at[pl.ds(0, sz)],
                            acc.at[pl.ds(s * SNPT + off, sz)])
            off += sz

    @pl.when(s == 15)
    def _():
        off = 0
        for sz in (128, 128, 128, 128, 8):
            pltpu.sync_copy(buf_a.at[pl.ds(0, sz)],
                            acc.at[pl.ds(15 * SNPT + off, sz)])
            off += sz

    plsc.subcore_barrier()

    def _pipe(gref):
        def stage_in(ci, jj):
            row = s * NCH + ci
            pltpu.async_copy(packed3d.at[pl.ds(row, 1)],
                             ring_p.at[pl.ds(jj, 1)], si[jj])
            pltpu.async_copy(ew3d.at[pl.ds(row, 1)],
                             ring_e.at[pl.ds(jj, 1)], si[jj])

        def wait_in(ci, jj):
            row = s * NCH + ci
            pltpu.make_async_copy(packed3d.at[pl.ds(row, 1)],
                                  ring_p.at[pl.ds(jj, 1)], si[jj]).wait()
            pltpu.make_async_copy(ew3d.at[pl.ds(row, 1)],
                                  ring_e.at[pl.ds(jj, 1)], si[jj]).wait()

        def unpack(jj):
            for k in range(CHP // 16):
                v = ring_p[jj, 0, pl.ds(k * 16, 16)]
                src_r[jj, pl.ds(k * 16, 16)] = jnp.bitwise_and(v, 0xFFFF)
                dst_r[jj, pl.ds(k * 16, 16)] = jnp.right_shift(v, 16)

        def scale(jj):
            buf = bufs[jj]

            def _grp(gi, _):
                wv = ring_e[jj, 0, pl.ds(gi * 16, 16)]
                for rr in range(16):
                    w = wv[rr]
                    for j in range(H // 16):
                        buf[gi * 16 + rr, pl.ds(j * 16, 16)] = (
                            buf[gi * 16 + rr, pl.ds(j * 16, 16)] * w)
                return 0
            lax.fori_loop(0, CHP // 16, _grp, 0)

        def issue_g(jj):
            pltpu.async_copy(gref.at[src_r.at[jj]], bufs[jj], sg[jj])

        def wait_g(jj):
            pltpu.make_async_copy(
                gref.at[src_r.at[jj]], bufs[jj], sg[jj]).wait()

        def issue_s(jj):
            pltpu.sync_copy(bufs[jj], acc.at[dst_r.at[jj]], add=True)

        def wait_s(jj):
            pass

        def phase(ci, X, first=False, static_tail=None):
            nxt = (X + 1) % 3
            prv = (X + 2) % 3
            if not first:
                wait_s(nxt)
            if static_tail is None:
                wait_in(ci + 1, nxt)
                unpack(nxt)
                issue_g(nxt)
            elif static_tail is False:
                @pl.when(ci + 1 < NCH)
                def _():
                    wait_in(ci + 1, nxt)
                    unpack(nxt)
                    issue_g(nxt)
            wait_g(X)
            scale(X)
            issue_s(X)
            if static_tail is None:
                stage_in(ci + 2, prv)
            elif static_tail is False:
                @pl.when(ci + 2 < NCH)
                def _():
                    stage_in(ci + 2, prv)

        stage_in(0, 0)
        stage_in(1, 1)
        wait_in(0, 0)
        unpack(0)
        issue_g(0)
        phase(0, 0, first=True)
        phase(1, 1, first=True)

        def _trip(t, _):
            cb = 3 * t + 2
            phase(cb, 2, static_tail=False)
            phase(cb + 1, 0, static_tail=False)
            phase(cb + 2, 1, static_tail=False)
            return 0

        lax.fori_loop(0, (NCH - 2) // 3, _trip, 0)

    @pl.when(c == 0)
    def _():
        _pipe(g0)

    @pl.when(c == 1)
    def _():
        _pipe(g1)

    plsc.subcore_barrier()

    @pl.when(c == 0)
    def _():
        @pl.when(s < 15)
        def _():
            pltpu.sync_copy(acc.at[pl.ds(s * SNPT, SNPT)],
                            out0.at[pl.ds(s * SNPT, SNPT)])

        @pl.when(s == 15)
        def _():
            pltpu.sync_copy(acc.at[pl.ds(15 * SNPT, SNPL)],
                            out0.at[pl.ds(15 * SNPT, SNPL)])

    @pl.when(c == 1)
    def _():
        @pl.when(s < 15)
        def _():
            pltpu.sync_copy(acc.at[pl.ds(s * SNPT, SNPT)],
                            out1.at[pl.ds(s * SNPT, SNPT)])

        @pl.when(s == 15)
        def _():
            pltpu.sync_copy(acc.at[pl.ds(15 * SNPT, SNPL)],
                            out1.at[pl.ds(15 * SNPT, SNPL)])


_seg_call = functools.partial(
    pl.kernel,
    out_type=[jax.ShapeDtypeStruct((N, H), _f32),
              jax.ShapeDtypeStruct((N, H), _f32)],
    mesh=_mesh,
    scratch_types=[
        pltpu.VMEM((3, 1, CHP), _i32),
        pltpu.VMEM((3, 1, CHP), _f32),
        pltpu.VMEM((3, CHP), _i32),
        pltpu.VMEM((3, CHP), _i32),
        pltpu.VMEM((CHP, H), _f32),
        pltpu.VMEM((CHP, H), _f32),
        pltpu.VMEM((CHP, H), _f32),
        pltpu.VMEM_SHARED((N, H), _f32),
    ] + [pltpu.SemaphoreType.DMA] * 9,
)(_seg_body)


_BLK = 1000
_GRID = N // _BLK
_dims = (((1,), (1,)), ((), ()))


def _mm_a_body(x_ref, w_ref, deg_ref, g0_ref, g1_ref, dis_ref):
    d = deg_ref[...] + 1.0
    dis = jnp.where(d > 0, lax.rsqrt(d), 0.0)
    h = lax.dot_general(x_ref[...], w_ref[...], _dims,
                        preferred_element_type=_f32)
    g = h * dis
    g0_ref[...] = g[:, :H]
    g1_ref[...] = g[:, H:]
    dis_ref[...] = dis


def _mm_a(x, w, degcol):
    return pl.pallas_call(
        _mm_a_body,
        grid=(_GRID,),
        in_specs=[
            pl.BlockSpec((_BLK, D), lambda i: (i, 0)),
            pl.BlockSpec((D, D), lambda i: (0, 0)),
            pl.BlockSpec((_BLK, 1), lambda i: (i, 0)),
        ],
        out_specs=[
            pl.BlockSpec((_BLK, H), lambda i: (i, 0)),
            pl.BlockSpec((_BLK, H), lambda i: (i, 0)),
            pl.BlockSpec((_BLK, 1), lambda i: (i, 0)),
        ],
        out_shape=[
            jax.ShapeDtypeStruct((N, H), _f32),
            jax.ShapeDtypeStruct((N, H), _f32),
            jax.ShapeDtypeStruct((N, 1), _f32),
        ],
    )(x, w, degcol)


def _mm_b_body(s0_ref, s1_ref, g0_ref, g1_ref, dis_ref, b_ref, w_ref,
               o0_ref, o1_ref):
    dis = dis_ref[...]
    sv = jnp.concatenate([s0_ref[...], s1_ref[...]], axis=1)
    gv = jnp.concatenate([g0_ref[...], g1_ref[...]], axis=1)
    hin = jnp.maximum(dis * (sv + gv) + b_ref[...], 0.0)
    g = lax.dot_general(hin, w_ref[...], _dims,
                        preferred_element_type=_f32) * dis
    o0_ref[...] = g[:, :H]
    o1_ref[...] = g[:, H:]


def _mm_b(s0, s1, g0, g1, dis, b2d, w):
    return pl.pallas_call(
        _mm_b_body,
        grid=(_GRID,),
        in_specs=[
            pl.BlockSpec((_BLK, H), lambda i: (i, 0)),
            pl.BlockSpec((_BLK, H), lambda i: (i, 0)),
            pl.BlockSpec((_BLK, H), lambda i: (i, 0)),
            pl.BlockSpec((_BLK, H), lambda i: (i, 0)),
            pl.BlockSpec((_BLK, 1), lambda i: (i, 0)),
            pl.BlockSpec((1, D), lambda i: (0, 0)),
            pl.BlockSpec((D, D), lambda i: (0, 0)),
        ],
        out_specs=[
            pl.BlockSpec((_BLK, H), lambda i: (i, 0)),
            pl.BlockSpec((_BLK, H), lambda i: (i, 0)),
        ],
        out_shape=[
            jax.ShapeDtypeStruct((N, H), _f32),
            jax.ShapeDtypeStruct((N, H), _f32),
        ],
    )(s0, s1, g0, g1, dis, b2d, w)


def _fin_body(s10, s11, g10, g11, dis1, b1,
              s20, s21, g20, g21, dis2, b2,
              fca, fcb_w, fcb_b, out_ref, acc):
    i = pl.program_id(0)

    h1 = jnp.maximum(
        dis1[...] * (jnp.concatenate([s10[...], s11[...]], axis=1)
                     + jnp.concatenate([g10[...], g11[...]], axis=1))
        + b1[...], 0.0)
    h2 = jnp.maximum(
        dis2[...] * (jnp.concatenate([s20[...], s21[...]], axis=1)
                     + jnp.concatenate([g20[...], g21[...]], axis=1))
        + b2[...], 0.0)
    c1 = jnp.sum(h1, axis=0, keepdims=True)
    c2 = jnp.sum(h2, axis=0, keepdims=True)

    @pl.when(i == 0)
    def _():
        acc[0:1, :] = c1
        acc[1:2, :] = c2

    @pl.when(i > 0)
    def _():
        acc[0:1, :] = acc[0:1, :] + c1
        acc[1:2, :] = acc[1:2, :] + c2

    @pl.when(i == _GRID - 1)
    def _():
        m1 = acc[0:1, :] * (1.0 / N)
        m2 = acc[1:2, :] * (1.0 / N)
        z = (jnp.sum(m1 * fca[...]) + jnp.sum(m2 * fcb_w[...])
             + fcb_b[0, 0])
        out_ref[...] = jax.nn.sigmoid(z) * jnp.ones((1, 1), _f32)


def _final(s10, s11, g10, g11, dis1, b1,
           s20, s21, g20, g21, dis2, b2, fca, fcbw, fcbb):
    blk = [
        pl.BlockSpec((_BLK, H), lambda i: (i, 0)),
        pl.BlockSpec((_BLK, H), lambda i: (i, 0)),
        pl.BlockSpec((_BLK, H), lambda i: (i, 0)),
        pl.BlockSpec((_BLK, H), lambda i: (i, 0)),
        pl.BlockSpec((_BLK, 1), lambda i: (i, 0)),
        pl.BlockSpec((1, D), lambda i: (0, 0)),
    ]
    return pl.pallas_call(
        _fin_body,
        grid=(_GRID,),
        in_specs=blk + blk + [
            pl.BlockSpec((1, D), lambda i: (0, 0)),
            pl.BlockSpec((1, D), lambda i: (0, 0)),
            pl.BlockSpec((1, 1), lambda i: (0, 0)),
        ],
        out_specs=pl.BlockSpec((1, 1), lambda i: (0, 0)),
        out_shape=jax.ShapeDtypeStruct((1, 1), _f32),
        scratch_shapes=[pltpu.VMEM((8, D), _f32)],
    )(s10, s11, g10, g11, dis1, b1,
      s20, s21, g20, g21, dis2, b2, fca, fcbw, fcbb)


def _tower_pre(edge_index, edge_weight):
    dst2d = edge_index[1].reshape(ROWS2D, CH)
    ew2d = edge_weight.reshape(ROWS2D, CH)
    npd = ROWS2D * (CHP - CH)
    spad = (jnp.arange(npd, dtype=jnp.int32) % N).reshape(ROWS2D, CHP - CH)
    dpad = ((jnp.arange(npd, dtype=jnp.int32) * 97) % N).reshape(
        ROWS2D, CHP - CH)
    srcp = jnp.concatenate([edge_index[0].reshape(ROWS2D, CH), spad], axis=1)
    dstp = jnp.concatenate([dst2d, dpad], axis=1)
    packed3d = (srcp | (dstp << 16)).reshape(ROWS2D, 1, CHP)
    ew3d = jnp.concatenate(
        [ew2d, jnp.zeros((ROWS2D, CHP - CH), jnp.float32)],
        axis=1).reshape(ROWS2D, 1, CHP)
    return dst2d, ew2d, packed3d, ew3d


def kernel(x1, edge_index1, edge_weight1, x2, edge_index2, edge_weight2,
           W1a, b1a, W1b, b1b, W2a, b2a, W2b, b2b, fcW, fcb):
    dst1, ew1, pk1, ewp1 = _tower_pre(edge_index1, edge_weight1)
    dst2, ew2, pk2, ewp2 = _tower_pre(edge_index2, edge_weight2)

    degs = _deg_call(dst1, ew1, dst2, ew2)
    deg1 = degs[0, :N].reshape(N, 1)
    deg2 = degs[1, :N].reshape(N, 1)

    g1a0, g1a1, dis1 = _mm_a(x1, W1a, deg1)
    g2a0, g2a1, dis2 = _mm_a(x2, W2a, deg2)

    s1a0, s1a1 = _seg_call(pk1, ewp1, g1a0, g1a1)
    s2a0, s2a1 = _seg_call(pk2, ewp2, g2a0, g2a1)

    g1b0, g1b1 = _mm_b(s1a0, s1a1, g1a0, g1a1, dis1, b1a.reshape(1, D), W1b)
    g2b0, g2b1 = _mm_b(s2a0, s2a1, g2a0, g2a1, dis2, b2a.reshape(1, D), W2b)

    s1b0, s1b1 = _seg_call(pk1, ewp1, g1b0, g1b1)
    s2b0, s2b1 = _seg_call(pk2, ewp2, g2b0, g2b1)

    return _final(
        s1b0, s1b1, g1b0, g1b1, dis1, b1b.reshape(1, D),
        s2b0, s2b1, g2b0, g2b1, dis2, b2b.reshape(1, D),
        fcW[:, :D], fcW[:, D:], fcb.reshape(1, 1))


# speedup vs baseline: 1.1963x; 1.1963x over previous
import functools

import jax
import jax.numpy as jnp
from jax import lax
from jax.experimental import pallas as pl
from jax.experimental.pallas import tpu as pltpu
from jax.experimental.pallas import tpu_sc as plsc

N = 10000
E = 160000
D = 256
H = 128
TECS = 16
EPT = E // TECS
CH = 125
CHP = 128
NCH = EPT // CH
ROWS2D = E // CH
SNPT = 632
SNPL = N - 15 * SNPT
NPAD = 10240
DPT = NPAD // TECS

_mesh = plsc.VectorSubcoreMesh(core_axis_name="c", subcore_axis_name="s")

_f32 = jnp.float32
_i32 = jnp.int32


def _deg_body(dst1, ew1, dst2, ew2, out, dst_v, ew_v, zbuf, acc):
    c = lax.axis_index("c")
    s = lax.axis_index("s")

    @pl.when(c == 0)
    def _():
        pltpu.sync_copy(dst1.at[pl.ds(s * NCH, NCH)], dst_v)
        pltpu.sync_copy(ew1.at[pl.ds(s * NCH, NCH)], ew_v)

    @pl.when(c == 1)
    def _():
        pltpu.sync_copy(dst2.at[pl.ds(s * NCH, NCH)], dst_v)
        pltpu.sync_copy(ew2.at[pl.ds(s * NCH, NCH)], ew_v)

    def _zero(i, _):
        zbuf[pl.ds(i * 16, 16)] = jnp.zeros((16,), _f32)
        return 0

    lax.fori_loop(0, DPT // 16, _zero, 0)
    pltpu.sync_copy(zbuf, acc.at[pl.ds(s * DPT, DPT)])
    plsc.subcore_barrier()

    def _chunk(ci, _):
        pltpu.sync_copy(ew_v.at[ci], acc.at[dst_v.at[ci]], add=True)
        return 0

    lax.fori_loop(0, NCH, _chunk, 0)
    plsc.subcore_barrier()

    @pl.when(c == 0)
    def _():
        pltpu.sync_copy(acc.at[pl.ds(s * DPT, DPT)], out.at[0, pl.ds(s * DPT, DPT)])

    @pl.when(c == 1)
    def _():
        pltpu.sync_copy(acc.at[pl.ds(s * DPT, DPT)], out.at[1, pl.ds(s * DPT, DPT)])


_deg_call = functools.partial(
    pl.kernel,
    out_type=jax.ShapeDtypeStruct((2, NPAD), _f32),
    mesh=_mesh,
    scratch_types=[
        pltpu.VMEM((NCH, CH), _i32),
        pltpu.VMEM((NCH, CH), _f32),
        pltpu.VMEM((DPT,), _f32),
        pltpu.VMEM_SHARED((NPAD,), _f32),
    ],
)(_deg_body)


def _seg_body(packed3d, ew3d, g0, g1, out0, out1,
              ring_p, ring_e, src_r, dst_r, buf_a, buf_b, buf_c, acc,
              sg0, sg1, sg2, ss0, ss1, ss2, si0, si1, si2):
    c = lax.axis_index("c")
    s = lax.axis_index("s")
    bufs = (buf_a, buf_b, buf_c)
    sg = (sg0, sg1, sg2)
    ss = (ss0, ss1, ss2)
    si = (si0, si1, si2)

    def _zrow(r, _):
        for j in range(H // 16):
            buf_a[r, pl.ds(j * 16, 16)] = jnp.zeros((16,), _f32)
        return 0

    lax.fori_loop(0, CHP, _zrow, 0)

    @pl.when(s < 15)
    def _():
        off = 0
        for sz in (128, 128, 128, 128, 120):
            pltpu.sync_copy(buf_a.at[pl.ds(0, sz)],
                            acc.at[pl.ds(s * SNPT + off, sz)])
            off += sz

    @pl.when(s == 15)
    def _():
        off = 0
        for sz in (128, 128, 128, 128, 8):
            pltpu.sync_copy(buf_a.at[pl.ds(0, sz)],
                            acc.at[pl.ds(15 * SNPT + off, sz)])
            off += sz

    plsc.subcore_barrier()

    def _pipe(gref):
        def stage_in(ci, jj):
            row = s * NCH + ci
            pltpu.async_copy(packed3d.at[pl.ds(row, 1)],
                             ring_p.at[pl.ds(jj, 1)], si[jj])
            pltpu.async_copy(ew3d.at[pl.ds(row, 1)],
                             ring_e.at[pl.ds(jj, 1)], si[jj])

        def wait_in(ci, jj):
            row = s * NCH + ci
            pltpu.make_async_copy(packed3d.at[pl.ds(row, 1)],
                                  ring_p.at[pl.ds(jj, 1)], si[jj]).wait()
            pltpu.make_async_copy(ew3d.at[pl.ds(row, 1)],
                                  ring_e.at[pl.ds(jj, 1)], si[jj]).wait()

        def unpack(jj):
            for k in range(CHP // 16):
                v = ring_p[jj, 0, pl.ds(k * 16, 16)]
                src_r[jj, pl.ds(k * 16, 16)] = jnp.bitwise_and(v, 0xFFFF)
                dst_r[jj, pl.ds(k * 16, 16)] = jnp.right_shift(v, 16)

        def scale(jj):
            buf = bufs[jj]

            def _grp(gi, _):
                wv = ring_e[jj, 0, pl.ds(gi * 16, 16)]
                for rr in range(16):
                    w = wv[rr]
                    for j in range(H // 16):
                        buf[gi * 16 + rr, pl.ds(j * 16, 16)] = (
                            buf[gi * 16 + rr, pl.ds(j * 16, 16)] * w)
                return 0
            lax.fori_loop(0, CHP // 16, _grp, 0)

        def issue_g(jj):
            pltpu.async_copy(gref.at[src_r.at[jj]], bufs[jj], sg[jj])

        def wait_g(jj):
            pltpu.make_async_copy(
                gref.at[src_r.at[jj]], bufs[jj], sg[jj]).wait()

        def issue_s(jj):
            pltpu.async_copy(bufs[jj], acc.at[dst_r.at[jj]], ss[jj],
                             add=True)

        def wait_s(jj):
            pltpu.make_async_copy(bufs[jj], acc.at[dst_r.at[jj]],
                                  ss[jj]).wait()

        def phase(ci, X, first=False, static_tail=None):
            nxt = (X + 1) % 3
            prv = (X + 2) % 3
            if static_tail is None:
                wait_in(ci + 1, nxt)
                unpack(nxt)
                issue_g(nxt)
            elif static_tail is False:
                @pl.when(ci + 1 < NCH)
                def _():
                    wait_in(ci + 1, nxt)
                    unpack(nxt)
                    issue_g(nxt)
            wait_g(X)
            scale(X)
            if not first:
                wait_s(prv)
            issue_s(X)
            if static_tail is None:
                stage_in(ci + 2, prv)
            elif static_tail is False:
                @pl.when(ci + 2 < NCH)
                def _():
                    stage_in(ci + 2, prv)

        stage_in(0, 0)
        stage_in(1, 1)
        wait_in(0, 0)
        unpack(0)
        issue_g(0)
        phase(0, 0, first=True)
        phase(1, 1)

        def _trip(t, _):
            cb = 3 * t + 2
            phase(cb, 2, static_tail=False)
            phase(cb + 1, 0, static_tail=False)
            phase(cb + 2, 1, static_tail=False)
            return 0

        lax.fori_loop(0, (NCH - 2) // 3, _trip, 0)
        wait_s(1)

    @pl.when(c == 0)
    def _():
        _pipe(g0)

    @pl.when(c == 1)
    def _():
        _pipe(g1)

    plsc.subcore_barrier()

    @pl.when(c == 0)
    def _():
        @pl.when(s < 15)
        def _():
            pltpu.sync_copy(acc.at[pl.ds(s * SNPT, SNPT)],
                            out0.at[pl.ds(s * SNPT, SNPT)])

        @pl.when(s == 15)
        def _():
            pltpu.sync_copy(acc.at[pl.ds(15 * SNPT, SNPL)],
                            out0.at[pl.ds(15 * SNPT, SNPL)])

    @pl.when(c == 1)
    def _():
        @pl.when(s < 15)
        def _():
            pltpu.sync_copy(acc.at[pl.ds(s * SNPT, SNPT)],
                            out1.at[pl.ds(s * SNPT, SNPT)])

        @pl.when(s == 15)
        def _():
            pltpu.sync_copy(acc.at[pl.ds(15 * SNPT, SNPL)],
                            out1.at[pl.ds(15 * SNPT, SNPL)])


_seg_call = functools.partial(
    pl.kernel,
    out_type=[jax.ShapeDtypeStruct((N, H), _f32),
              jax.ShapeDtypeStruct((N, H), _f32)],
    mesh=_mesh,
    scratch_types=[
        pltpu.VMEM((3, 1, CHP), _i32),
        pltpu.VMEM((3, 1, CHP), _f32),
        pltpu.VMEM((3, CHP), _i32),
        pltpu.VMEM((3, CHP), _i32),
        pltpu.VMEM((CHP, H), _f32),
        pltpu.VMEM((CHP, H), _f32),
        pltpu.VMEM((CHP, H), _f32),
        pltpu.VMEM_SHARED((N, H), _f32),
    ] + [pltpu.SemaphoreType.DMA] * 9,
)(_seg_body)


_BLK = 1000
_GRID = N // _BLK
_dims = (((1,), (1,)), ((), ()))


def _mm_a_body(x_ref, w_ref, deg_ref, g0_ref, g1_ref, dis_ref):
    d = deg_ref[...] + 1.0
    dis = jnp.where(d > 0, lax.rsqrt(d), 0.0)
    h = lax.dot_general(x_ref[...], w_ref[...], _dims,
                        preferred_element_type=_f32)
    g = h * dis
    g0_ref[...] = g[:, :H]
    g1_ref[...] = g[:, H:]
    dis_ref[...] = dis


def _mm_a(x, w, degcol):
    return pl.pallas_call(
        _mm_a_body,
        grid=(_GRID,),
        in_specs=[
            pl.BlockSpec((_BLK, D), lambda i: (i, 0)),
            pl.BlockSpec((D, D), lambda i: (0, 0)),
            pl.BlockSpec((_BLK, 1), lambda i: (i, 0)),
        ],
        out_specs=[
            pl.BlockSpec((_BLK, H), lambda i: (i, 0)),
            pl.BlockSpec((_BLK, H), lambda i: (i, 0)),
            pl.BlockSpec((_BLK, 1), lambda i: (i, 0)),
        ],
        out_shape=[
            jax.ShapeDtypeStruct((N, H), _f32),
            jax.ShapeDtypeStruct((N, H), _f32),
            jax.ShapeDtypeStruct((N, 1), _f32),
        ],
    )(x, w, degcol)


def _mm_b_body(s0_ref, s1_ref, g0_ref, g1_ref, dis_ref, b_ref, w_ref,
               o0_ref, o1_ref):
    dis = dis_ref[...]
    sv = jnp.concatenate([s0_ref[...], s1_ref[...]], axis=1)
    gv = jnp.concatenate([g0_ref[...], g1_ref[...]], axis=1)
    hin = jnp.maximum(dis * (sv + gv) + b_ref[...], 0.0)
    g = lax.dot_general(hin, w_ref[...], _dims,
                        preferred_element_type=_f32) * dis
    o0_ref[...] = g[:, :H]
    o1_ref[...] = g[:, H:]


def _mm_b(s0, s1, g0, g1, dis, b2d, w):
    return pl.pallas_call(
        _mm_b_body,
        grid=(_GRID,),
        in_specs=[
            pl.BlockSpec((_BLK, H), lambda i: (i, 0)),
            pl.BlockSpec((_BLK, H), lambda i: (i, 0)),
            pl.BlockSpec((_BLK, H), lambda i: (i, 0)),
            pl.BlockSpec((_BLK, H), lambda i: (i, 0)),
            pl.BlockSpec((_BLK, 1), lambda i: (i, 0)),
            pl.BlockSpec((1, D), lambda i: (0, 0)),
            pl.BlockSpec((D, D), lambda i: (0, 0)),
        ],
        out_specs=[
            pl.BlockSpec((_BLK, H), lambda i: (i, 0)),
            pl.BlockSpec((_BLK, H), lambda i: (i, 0)),
        ],
        out_shape=[
            jax.ShapeDtypeStruct((N, H), _f32),
            jax.ShapeDtypeStruct((N, H), _f32),
        ],
    )(s0, s1, g0, g1, dis, b2d, w)


def _fin_body(s10, s11, g10, g11, dis1, b1,
              s20, s21, g20, g21, dis2, b2,
              fca, fcb_w, fcb_b, out_ref, acc):
    i = pl.program_id(0)

    h1 = jnp.maximum(
        dis1[...] * (jnp.concatenate([s10[...], s11[...]], axis=1)
                     + jnp.concatenate([g10[...], g11[...]], axis=1))
        + b1[...], 0.0)
    h2 = jnp.maximum(
        dis2[...] * (jnp.concatenate([s20[...], s21[...]], axis=1)
                     + jnp.concatenate([g20[...], g21[...]], axis=1))
        + b2[...], 0.0)
    c1 = jnp.sum(h1, axis=0, keepdims=True)
    c2 = jnp.sum(h2, axis=0, keepdims=True)

    @pl.when(i == 0)
    def _():
        acc[0:1, :] = c1
        acc[1:2, :] = c2

    @pl.when(i > 0)
    def _():
        acc[0:1, :] = acc[0:1, :] + c1
        acc[1:2, :] = acc[1:2, :] + c2

    @pl.when(i == _GRID - 1)
    def _():
        m1 = acc[0:1, :] * (1.0 / N)
        m2 = acc[1:2, :] * (1.0 / N)
        z = (jnp.sum(m1 * fca[...]) + jnp.sum(m2 * fcb_w[...])
             + fcb_b[0, 0])
        out_ref[...] = jax.nn.sigmoid(z) * jnp.ones((1, 1), _f32)


def _final(s10, s11, g10, g11, dis1, b1,
           s20, s21, g20, g21, dis2, b2, fca, fcbw, fcbb):
    blk = [
        pl.BlockSpec((_BLK, H), lambda i: (i, 0)),
        pl.BlockSpec((_BLK, H), lambda i: (i, 0)),
        pl.BlockSpec((_BLK, H), lambda i: (i, 0)),
        pl.BlockSpec((_BLK, H), lambda i: (i, 0)),
        pl.BlockSpec((_BLK, 1), lambda i: (i, 0)),
        pl.BlockSpec((1, D), lambda i: (0, 0)),
    ]
    return pl.pallas_call(
        _fin_body,
        grid=(_GRID,),
        in_specs=blk + blk + [
            pl.BlockSpec((1, D), lambda i: (0, 0)),
            pl.BlockSpec((1, D), lambda i: (0, 0)),
            pl.BlockSpec((1, 1), lambda i: (0, 0)),
        ],
        out_specs=pl.BlockSpec((1, 1), lambda i: (0, 0)),
        out_shape=jax.ShapeDtypeStruct((1, 1), _f32),
        scratch_shapes=[pltpu.VMEM((8, D), _f32)],
    )(s10, s11, g10, g11, dis1, b1,
      s20, s21, g20, g21, dis2, b2, fca, fcbw, fcbb)


def _tower_pre(edge_index, edge_weight):
    dst2d = edge_index[1].reshape(ROWS2D, CH)
    ew2d = edge_weight.reshape(ROWS2D, CH)
    npd = ROWS2D * (CHP - CH)
    spad = (jnp.arange(npd, dtype=jnp.int32) % N).reshape(ROWS2D, CHP - CH)
    dpad = ((jnp.arange(npd, dtype=jnp.int32) * 97) % N).reshape(
        ROWS2D, CHP - CH)
    srcp = jnp.concatenate([edge_index[0].reshape(ROWS2D, CH), spad], axis=1)
    dstp = jnp.concatenate([dst2d, dpad], axis=1)
    packed3d = (srcp | (dstp << 16)).reshape(ROWS2D, 1, CHP)
    ew3d = jnp.concatenate(
        [ew2d, jnp.zeros((ROWS2D, CHP - CH), jnp.float32)],
        axis=1).reshape(ROWS2D, 1, CHP)
    return dst2d, ew2d, packed3d, ew3d


def kernel(x1, edge_index1, edge_weight1, x2, edge_index2, edge_weight2,
           W1a, b1a, W1b, b1b, W2a, b2a, W2b, b2b, fcW, fcb):
    dst1, ew1, pk1, ewp1 = _tower_pre(edge_index1, edge_weight1)
    dst2, ew2, pk2, ewp2 = _tower_pre(edge_index2, edge_weight2)

    degs = _deg_call(dst1, ew1, dst2, ew2)
    deg1 = degs[0, :N].reshape(N, 1)
    deg2 = degs[1, :N].reshape(N, 1)

    g1a0, g1a1, dis1 = _mm_a(x1, W1a, deg1)
    g2a0, g2a1, dis2 = _mm_a(x2, W2a, deg2)

    s1a0, s1a1 = _seg_call(pk1, ewp1, g1a0, g1a1)
    s2a0, s2a1 = _seg_call(pk2, ewp2, g2a0, g2a1)

    g1b0, g1b1 = _mm_b(s1a0, s1a1, g1a0, g1a1, dis1, b1a.reshape(1, D), W1b)
    g2b0, g2b1 = _mm_b(s2a0, s2a1, g2a0, g2a1, dis2, b2a.reshape(1, D), W2b)

    s1b0, s1b1 = _seg_call(pk1, ewp1, g1b0, g1b1)
    s2b0, s2b1 = _seg_call(pk2, ewp2, g2b0, g2b1)

    return _final(
        s1b0, s1b1, g1b0, g1b1, dis1, b1b.reshape(1, D),
        s2b0, s2b1, g2b0, g2b1, dis2, b2b.reshape(1, D),
        fcW[:, :D], fcW[:, D:], fcb.reshape(1, 1))
